# Initial kernel scaffold; baseline (speedup 1.0000x reference)
import jax
import jax.numpy as jnp
from jax.experimental import pallas as pl


def kernel(hidden_states, gate_weight, w13_weight, w13_weight_scale, w2_weight, w2_weight_scale):
    raise NotImplementedError("write your pallas kernel here")


# speedup vs baseline: 1.5877x; 1.5877x over previous
import functools

import jax
import jax.numpy as jnp
from jax.experimental import pallas as pl
from jax.experimental.pallas import tpu as pltpu

B, S, H, I, E, K = 1, 2048, 2048, 4096, 8, 2
N = B * S

T = 512
C = 1024
NI = I // C
NT = N // T


def _moe_kernel(x_ref, gw_ref, w13g_ref, w13u_ref, s13g_ref, s13u_ref,
                w2_ref, s2w_ref, out_ref,
                q1_s, s1_s, wts_s, act_s, acc_s):
    e = pl.program_id(1)
    i = pl.program_id(2)

    @pl.when(jnp.logical_and(e == 0, i == 0))
    def _prologue():
        x = x_ref[...]
        logits = jax.lax.dot_general(
            x, gw_ref[...], (((1,), (1,)), ((), ())),
            preferred_element_type=jnp.float32)
        idx = jax.lax.broadcasted_iota(jnp.int32, (T, E), 1)
        m1 = jnp.max(logits, axis=1, keepdims=True)
        i1 = jnp.min(jnp.where(logits == m1, idx, E), axis=1, keepdims=True)
        masked = jnp.where(idx == i1, -jnp.inf, logits)
        m2 = jnp.max(masked, axis=1, keepdims=True)
        i2 = jnp.min(jnp.where(masked == m2, idx, E), axis=1, keepdims=True)
        b = jnp.exp(m2 - m1)
        denom = 1.0 + b
        g1 = 1.0 / denom
        g2 = b / denom
        wts_s[...] = jnp.where(idx == i1, g1, 0.0) + jnp.where(idx == i2, g2, 0.0)
        s1 = jnp.clip(jnp.max(jnp.abs(x), axis=1, keepdims=True), 1e-8, None) / 127.0
        s1_s[...] = s1
        q1_s[...] = jnp.clip(jnp.round(x / s1), -127.0, 127.0).astype(jnp.bfloat16)
        acc_s[...] = jnp.zeros_like(acc_s)

    q1 = q1_s[...]
    s1 = s1_s[...]
    hg = jax.lax.dot_general(
        q1, w13g_ref[0].astype(jnp.bfloat16), (((1,), (1,)), ((), ())),
        preferred_element_type=jnp.float32) * s1 * s13g_ref[0]
    hu = jax.lax.dot_general(
        q1, w13u_ref[0].astype(jnp.bfloat16), (((1,), (1,)), ((), ())),
        preferred_element_type=jnp.float32) * s1 * s13u_ref[0]
    act_s[:, pl.ds(i * C, C)] = hg * jax.lax.logistic(hg) * hu

    @pl.when(i == NI - 1)
    def _fc2():
        act = act_s[...]
        s2 = jnp.clip(jnp.max(jnp.abs(act), axis=1, keepdims=True), 1e-8, None) / 127.0
        q2 = jnp.clip(jnp.round(act / s2), -127.0, 127.0).astype(jnp.bfloat16)
        h2 = jax.lax.dot_general(
            q2, w2_ref[0].astype(jnp.bfloat16), (((1,), (1,)), ((), ())),
            preferred_element_type=jnp.float32) * s2 * s2w_ref[0]
        idx = jax.lax.broadcasted_iota(jnp.int32, (T, E), 1)
        wcol = jnp.sum(jnp.where(idx == e, wts_s[...], 0.0), axis=1, keepdims=True)
        acc_s[...] += wcol * h2

        @pl.when(e == E - 1)
        def _write():
            out_ref[...] = acc_s[...]


@functools.partial(jax.jit, static_argnums=())
def _run(x, gate_weight, w13_weight, w13_weight_scale, w2_weight, w2_weight_scale):
    grid = (NT, E, NI)
    out = pl.pallas_call(
        _moe_kernel,
        grid=grid,
        in_specs=[
            pl.BlockSpec((T, H), lambda t, e, i: (t, 0)),
            pl.BlockSpec((E, H), lambda t, e, i: (0, 0)),
            pl.BlockSpec((1, C, H), lambda t, e, i: (e, i, 0)),
            pl.BlockSpec((1, C, H), lambda t, e, i: (e, NI + i, 0)),
            pl.BlockSpec((1, 1, C), lambda t, e, i: (e * 2 * NI + i, 0, 0)),
            pl.BlockSpec((1, 1, C), lambda t, e, i: (e * 2 * NI + NI + i, 0, 0)),
            pl.BlockSpec((1, H, I), lambda t, e, i: (e, 0, 0)),
            pl.BlockSpec((1, 1, H), lambda t, e, i: (e, 0, 0)),
        ],
        out_specs=pl.BlockSpec((T, H), lambda t, e, i: (t, 0)),
        out_shape=jax.ShapeDtypeStruct((N, H), jnp.float32),
        scratch_shapes=[
            pltpu.VMEM((T, H), jnp.bfloat16),
            pltpu.VMEM((T, 1), jnp.float32),
            pltpu.VMEM((T, E), jnp.float32),
            pltpu.VMEM((T, I), jnp.float32),
            pltpu.VMEM((T, H), jnp.float32),
        ],
        compiler_params=pltpu.CompilerParams(
            dimension_semantics=("parallel", "arbitrary", "arbitrary"),
            vmem_limit_bytes=64 * 1024 * 1024,
        ),
    )(x, gate_weight, w13_weight, w13_weight,
      w13_weight_scale.reshape(E * 2 * NI, 1, C),
      w13_weight_scale.reshape(E * 2 * NI, 1, C),
      w2_weight, w2_weight_scale.reshape(E, 1, H))
    return out


def kernel(hidden_states, gate_weight, w13_weight, w13_weight_scale, w2_weight, w2_weight_scale):
    b, s, h = hidden_states.shape
    x = hidden_states.reshape(-1, h)
    out = _run(x, gate_weight, w13_weight, w13_weight_scale, w2_weight, w2_weight_scale)
    return out.reshape(b, s, h)


# speedup vs baseline: 2.1432x; 1.3499x over previous
import jax
import jax.numpy as jnp
from jax.experimental import pallas as pl
from jax.experimental.pallas import tpu as pltpu

B, S, H, I, E, K = 1, 2048, 2048, 4096, 8, 2
N = B * S

TG = 256
NTILES = N * K // TG + E
NP = NTILES * TG
C = 512
NI = I // C
TN = 512
PT = 1024
HUGE = 1 << 22


def _gate_kernel(x_ref, gw_ref, q1_ref, swt_ref, rank_ref, counts_ref):
    x = x_ref[...]
    logits = jax.lax.dot_general(
        x, gw_ref[...], (((1,), (1,)), ((), ())),
        preferred_element_type=jnp.float32)
    idx = jax.lax.broadcasted_iota(jnp.int32, (N, E), 1)
    m1 = jnp.max(logits, axis=1, keepdims=True)
    i1 = jnp.min(jnp.where(logits == m1, idx, E), axis=1, keepdims=True)
    masked = jnp.where(idx == i1, -jnp.inf, logits)
    m2 = jnp.max(masked, axis=1, keepdims=True)
    i2 = jnp.min(jnp.where(masked == m2, idx, E), axis=1, keepdims=True)
    b = jnp.exp(m2 - m1)
    denom = 1.0 + b
    wts = (jnp.where(idx == i1, 1.0 / denom, 0.0)
           + jnp.where(idx == i2, b / denom, 0.0))
    s1 = jnp.clip(jnp.max(jnp.abs(x), axis=1, keepdims=True), 1e-8, None) / 127.0
    q1_ref[...] = jnp.clip(jnp.round(x / s1), -127.0, 127.0).astype(jnp.bfloat16)
    zeros7 = jnp.zeros((N, 16 - E - 1), jnp.float32)
    swt_ref[...] = jnp.concatenate([wts, s1, zeros7], axis=1)

    m = (wts > 0.0).astype(jnp.float32)
    mb = m.astype(jnp.bfloat16)
    ck = 512
    for k in range(N // ck):
        ri = jax.lax.broadcasted_iota(jnp.int32, (ck, N), 0) + k * ck
        ci = jax.lax.broadcasted_iota(jnp.int32, (ck, N), 1)
        lk = (ci <= ri).astype(jnp.bfloat16)
        cum = jax.lax.dot_general(
            lk, mb, (((1,), (0,)), ((), ())),
            preferred_element_type=jnp.float32)
        mrows = m[k * ck:(k + 1) * ck]
        rank = (cum - mrows).astype(jnp.int32)
        rank_ref[k * ck:(k + 1) * ck, :] = jnp.where(mrows > 0.0, rank, HUGE)
        if k == N // ck - 1:
            counts_ref[...] = cum[ck - 1:ck, :].astype(jnp.int32)


def _gemm_kernel(te_ref, act_ref, offp_ref,
                 q1_ref, rank_ref, swt_ref, w13g_ref, w13u_ref,
                 s13g_ref, s13u_ref, w2_ref, s2w_ref, h2w_ref,
                 pt_s, q1t_s, st_s, act_s):
    g = pl.program_id(0)
    i = pl.program_id(1)
    te = te_ref[g]
    active = act_ref[g]

    @pl.when(active > 0)
    def _work():
        @pl.when(i == 0)
        def _gather():
            p0 = g * TG
            ci = jax.lax.broadcasted_iota(jnp.int32, (N, TG), 1) + p0
            acc = jnp.zeros((N, TG), jnp.float32)
            for e in range(E):
                pos_e = rank_ref[:, e:e + 1] + offp_ref[e]
                acc += (pos_e == ci).astype(jnp.float32)
            pt = acc.astype(jnp.bfloat16)
            pt_s[...] = pt
            q1t = jax.lax.dot_general(
                pt, q1_ref[...], (((0,), (0,)), ((), ())),
                preferred_element_type=jnp.float32)
            q1t_s[...] = q1t.astype(jnp.bfloat16)
            st_s[...] = jax.lax.dot_general(
                acc, swt_ref[...], (((0,), (0,)), ((), ())),
                precision=jax.lax.Precision.HIGHEST,
                preferred_element_type=jnp.float32)

        q1t = q1t_s[...]
        s1t = st_s[:, E:E + 1]
        hg = jax.lax.dot_general(
            q1t, w13g_ref[0].astype(jnp.bfloat16), (((1,), (1,)), ((), ())),
            preferred_element_type=jnp.float32) * s1t * s13g_ref[0]
        hu = jax.lax.dot_general(
            q1t, w13u_ref[0].astype(jnp.bfloat16), (((1,), (1,)), ((), ())),
            preferred_element_type=jnp.float32) * s1t * s13u_ref[0]
        act_s[:, pl.ds(i * C, C)] = hg * jax.lax.logistic(hg) * hu

        @pl.when(i == NI - 1)
        def _fc2():
            act = act_s[...]
            s2 = jnp.clip(jnp.max(jnp.abs(act), axis=1, keepdims=True),
                          1e-8, None) / 127.0
            q2 = jnp.clip(jnp.round(act / s2), -127.0, 127.0).astype(jnp.bfloat16)
            h2 = jax.lax.dot_general(
                q2, w2_ref[0].astype(jnp.bfloat16), (((1,), (1,)), ((), ())),
                preferred_element_type=jnp.float32) * s2 * s2w_ref[0]
            lidx = jax.lax.broadcasted_iota(jnp.int32, (TG, 16), 1)
            wcol = jnp.sum(jnp.where(lidx == te, st_s[...], 0.0),
                           axis=1, keepdims=True)
            h2w_ref[...] = (wcol * h2).astype(jnp.bfloat16)

    @pl.when(jnp.logical_and(active == 0, i == NI - 1))
    def _zero():
        h2w_ref[...] = jnp.zeros_like(h2w_ref)


def _combine_kernel(offp_ref, rank_ref, h2w_ref, out_ref):
    p = pl.program_id(1)
    p0 = p * PT
    ci = jax.lax.broadcasted_iota(jnp.int32, (TN, PT), 1) + p0
    acc = jnp.zeros((TN, PT), jnp.float32)
    for e in range(E):
        pos_e = rank_ref[:, e:e + 1] + offp_ref[e]
        acc += (pos_e == ci).astype(jnp.float32)
    gmat = acc.astype(jnp.bfloat16)
    contrib = jax.lax.dot_general(
        gmat, h2w_ref[...], (((1,), (0,)), ((), ())),
        preferred_element_type=jnp.float32)

    @pl.when(p == 0)
    def _init():
        out_ref[...] = contrib

    @pl.when(p != 0)
    def _acc():
        out_ref[...] += contrib


@jax.jit
def _run(x, gate_weight, w13_weight, w13_weight_scale, w2_weight, w2_weight_scale):
    q1, swt, rank, counts = pl.pallas_call(
        _gate_kernel,
        out_shape=[
            jax.ShapeDtypeStruct((N, H), jnp.bfloat16),
            jax.ShapeDtypeStruct((N, 16), jnp.float32),
            jax.ShapeDtypeStruct((N, E), jnp.int32),
            jax.ShapeDtypeStruct((1, E), jnp.int32),
        ],
        compiler_params=pltpu.CompilerParams(
            vmem_limit_bytes=64 * 1024 * 1024,
        ),
    )(x, gate_weight)

    cnt = counts.reshape(E)
    padded = ((cnt + TG - 1) // TG) * TG
    offp = jnp.concatenate([jnp.zeros((1,), jnp.int32),
                            jnp.cumsum(padded).astype(jnp.int32)])
    total = offp[E]
    gstart = jnp.arange(NTILES, dtype=jnp.int32) * TG
    te_raw = jnp.sum((offp[None, :E] <= gstart[:, None]).astype(jnp.int32),
                     axis=1) - 1
    te_last = jnp.sum((offp[:E] <= total - 1).astype(jnp.int32)) - 1
    act_fl = (gstart < total).astype(jnp.int32)
    te = jnp.where(act_fl > 0, te_raw, te_last)
    offp8 = offp[:E]

    grid_spec = pltpu.PrefetchScalarGridSpec(
        num_scalar_prefetch=3,
        grid=(NTILES, NI),
        in_specs=[
            pl.BlockSpec((N, H), lambda g, i, te, af, op: (0, 0)),
            pl.BlockSpec((N, E), lambda g, i, te, af, op: (0, 0)),
            pl.BlockSpec((N, 16), lambda g, i, te, af, op: (0, 0)),
            pl.BlockSpec((1, C, H), lambda g, i, te, af, op: (te[g], i, 0)),
            pl.BlockSpec((1, C, H), lambda g, i, te, af, op: (te[g], NI + i, 0)),
            pl.BlockSpec((1, 1, C), lambda g, i, te, af, op: (te[g] * 2 * NI + i, 0, 0)),
            pl.BlockSpec((1, 1, C), lambda g, i, te, af, op: (te[g] * 2 * NI + NI + i, 0, 0)),
            pl.BlockSpec((1, H, I), lambda g, i, te, af, op: (te[g], 0, 0)),
            pl.BlockSpec((1, 1, H), lambda g, i, te, af, op: (te[g], 0, 0)),
        ],
        out_specs=pl.BlockSpec((TG, H), lambda g, i, te, af, op: (g, 0)),
        scratch_shapes=[
            pltpu.VMEM((N, TG), jnp.bfloat16),
            pltpu.VMEM((TG, H), jnp.bfloat16),
            pltpu.VMEM((TG, 16), jnp.float32),
            pltpu.VMEM((TG, I), jnp.float32),
        ],
    )
    h2w = pl.pallas_call(
        _gemm_kernel,
        grid_spec=grid_spec,
        out_shape=jax.ShapeDtypeStruct((NP, H), jnp.bfloat16),
        compiler_params=pltpu.CompilerParams(
            dimension_semantics=("arbitrary", "arbitrary"),
            vmem_limit_bytes=64 * 1024 * 1024,
        ),
    )(te, act_fl, offp8, q1, rank, swt,
      w13_weight, w13_weight,
      w13_weight_scale.reshape(E * 2 * NI, 1, C),
      w13_weight_scale.reshape(E * 2 * NI, 1, C),
      w2_weight, w2_weight_scale.reshape(E, 1, H))

    grid_spec_c = pltpu.PrefetchScalarGridSpec(
        num_scalar_prefetch=1,
        grid=(N // TN, NP // PT),
        in_specs=[
            pl.BlockSpec((TN, E), lambda n, p, op: (n, 0)),
            pl.BlockSpec((PT, H), lambda n, p, op: (p, 0)),
        ],
        out_specs=pl.BlockSpec((TN, H), lambda n, p, op: (n, 0)),
    )
    out = pl.pallas_call(
        _combine_kernel,
        grid_spec=grid_spec_c,
        out_shape=jax.ShapeDtypeStruct((N, H), jnp.float32),
        compiler_params=pltpu.CompilerParams(
            dimension_semantics=("arbitrary", "arbitrary"),
            vmem_limit_bytes=64 * 1024 * 1024,
        ),
    )(offp8, rank, h2w)
    return out


def kernel(hidden_states, gate_weight, w13_weight, w13_weight_scale, w2_weight, w2_weight_scale):
    b, s, h = hidden_states.shape
    x = hidden_states.reshape(-1, h)
    out = _run(x, gate_weight, w13_weight, w13_weight_scale,
               w2_weight, w2_weight_scale)
    return out.reshape(b, s, h)


# speedup vs baseline: 2.3302x; 1.0872x over previous
import jax
import jax.numpy as jnp
from jax.experimental import pallas as pl
from jax.experimental.pallas import tpu as pltpu

B, S, H, I, E, K = 1, 2048, 2048, 4096, 8, 2
N = B * S

TG = 256
NTILES = N * K // TG + E
NP = NTILES * TG
C = 1024
NI = I // C
TN = 512
PT = 2048
HUGE = 1 << 22


def _gate_kernel(x_ref, gw_ref, q1_ref, swt_ref, rank_ref, counts_ref):
    x = x_ref[...]
    logits = jax.lax.dot_general(
        x, gw_ref[...], (((1,), (1,)), ((), ())),
        preferred_element_type=jnp.float32)
    idx = jax.lax.broadcasted_iota(jnp.int32, (N, E), 1)
    m1 = jnp.max(logits, axis=1, keepdims=True)
    i1 = jnp.min(jnp.where(logits == m1, idx, E), axis=1, keepdims=True)
    masked = jnp.where(idx == i1, -jnp.inf, logits)
    m2 = jnp.max(masked, axis=1, keepdims=True)
    i2 = jnp.min(jnp.where(masked == m2, idx, E), axis=1, keepdims=True)
    b = jnp.exp(m2 - m1)
    denom = 1.0 + b
    wts = (jnp.where(idx == i1, 1.0 / denom, 0.0)
           + jnp.where(idx == i2, b / denom, 0.0))
    s1 = jnp.clip(jnp.max(jnp.abs(x), axis=1, keepdims=True), 1e-8, None) / 127.0
    q1_ref[...] = jnp.clip(jnp.round(x / s1), -127.0, 127.0).astype(jnp.bfloat16)
    zeros7 = jnp.zeros((N, 16 - E - 1), jnp.float32)
    swt_ref[...] = jnp.concatenate([wts, s1, zeros7], axis=1)

    m = (wts > 0.0).astype(jnp.float32)
    mb = m.astype(jnp.bfloat16)
    ck = 512
    for k in range(N // ck):
        ri = jax.lax.broadcasted_iota(jnp.int32, (ck, N), 0) + k * ck
        ci = jax.lax.broadcasted_iota(jnp.int32, (ck, N), 1)
        lk = (ci <= ri).astype(jnp.bfloat16)
        cum = jax.lax.dot_general(
            lk, mb, (((1,), (0,)), ((), ())),
            preferred_element_type=jnp.float32)
        mrows = m[k * ck:(k + 1) * ck]
        rank = (cum - mrows).astype(jnp.int32)
        rank_ref[k * ck:(k + 1) * ck, :] = jnp.where(mrows > 0.0, rank, HUGE)
        if k == N // ck - 1:
            counts_ref[...] = cum[ck - 1:ck, :].astype(jnp.int32)


def _gemm_kernel(te_ref, act_ref, offp_ref,
                 q1_ref, rank_ref, swt_ref, w13g_ref, w13u_ref,
                 s13g_ref, s13u_ref, w2_ref, s2w_ref, h2w_ref,
                 pt_s, q1t_s, st_s, act_s):
    g = pl.program_id(0)
    i = pl.program_id(1)
    te = te_ref[g]
    active = act_ref[g]

    @pl.when(active > 0)
    def _work():
        @pl.when(i == 0)
        def _gather():
            p0 = g * TG
            ci = jax.lax.broadcasted_iota(jnp.int32, (N, TG), 1) + p0
            acc = jnp.zeros((N, TG), jnp.float32)
            for e in range(E):
                pos_e = rank_ref[:, e:e + 1] + offp_ref[e]
                acc += (pos_e == ci).astype(jnp.float32)
            pt = acc.astype(jnp.bfloat16)
            pt_s[...] = pt
            q1t = jax.lax.dot_general(
                pt, q1_ref[...], (((0,), (0,)), ((), ())),
                preferred_element_type=jnp.float32)
            q1t_s[...] = q1t.astype(jnp.bfloat16)
            st_s[...] = jax.lax.dot_general(
                acc, swt_ref[...], (((0,), (0,)), ((), ())),
                precision=jax.lax.Precision.HIGHEST,
                preferred_element_type=jnp.float32)

        q1t = q1t_s[...]
        s1t = st_s[:, E:E + 1]
        hg = jax.lax.dot_general(
            q1t, w13g_ref[0].astype(jnp.bfloat16), (((1,), (1,)), ((), ())),
            preferred_element_type=jnp.float32) * s1t * s13g_ref[0]
        hu = jax.lax.dot_general(
            q1t, w13u_ref[0].astype(jnp.bfloat16), (((1,), (1,)), ((), ())),
            preferred_element_type=jnp.float32) * s1t * s13u_ref[0]
        act_s[:, pl.ds(i * C, C)] = hg * jax.lax.logistic(hg) * hu

        @pl.when(i == NI - 1)
        def _fc2():
            act = act_s[...]
            s2 = jnp.clip(jnp.max(jnp.abs(act), axis=1, keepdims=True),
                          1e-8, None) / 127.0
            q2 = jnp.clip(jnp.round(act / s2), -127.0, 127.0).astype(jnp.bfloat16)
            h2 = jax.lax.dot_general(
                q2, w2_ref[0].astype(jnp.bfloat16), (((1,), (1,)), ((), ())),
                preferred_element_type=jnp.float32) * s2 * s2w_ref[0]
            lidx = jax.lax.broadcasted_iota(jnp.int32, (TG, 16), 1)
            wcol = jnp.sum(jnp.where(lidx == te, st_s[...], 0.0),
                           axis=1, keepdims=True)
            h2w_ref[...] = (wcol * h2).astype(jnp.bfloat16)

    @pl.when(jnp.logical_and(active == 0, i == NI - 1))
    def _zero():
        h2w_ref[...] = jnp.zeros_like(h2w_ref)


def _combine_kernel(offp_ref, rank_ref, h2w_ref, out_ref):
    p = pl.program_id(1)
    p0 = p * PT
    ci = jax.lax.broadcasted_iota(jnp.int32, (TN, PT), 1) + p0
    acc = jnp.zeros((TN, PT), jnp.float32)
    for e in range(E):
        pos_e = rank_ref[:, e:e + 1] + offp_ref[e]
        acc += (pos_e == ci).astype(jnp.float32)
    gmat = acc.astype(jnp.bfloat16)
    contrib = jax.lax.dot_general(
        gmat, h2w_ref[...], (((1,), (0,)), ((), ())),
        preferred_element_type=jnp.float32)

    @pl.when(p == 0)
    def _init():
        out_ref[...] = contrib

    @pl.when(p != 0)
    def _acc():
        out_ref[...] += contrib


@jax.jit
def _run(x, gate_weight, w13_weight, w13_weight_scale, w2_weight, w2_weight_scale):
    q1, swt, rank, counts = pl.pallas_call(
        _gate_kernel,
        out_shape=[
            jax.ShapeDtypeStruct((N, H), jnp.bfloat16),
            jax.ShapeDtypeStruct((N, 16), jnp.float32),
            jax.ShapeDtypeStruct((N, E), jnp.int32),
            jax.ShapeDtypeStruct((1, E), jnp.int32),
        ],
        compiler_params=pltpu.CompilerParams(
            vmem_limit_bytes=64 * 1024 * 1024,
        ),
    )(x, gate_weight)

    cnt = counts.reshape(E)
    padded = ((cnt + TG - 1) // TG) * TG
    offp = jnp.concatenate([jnp.zeros((1,), jnp.int32),
                            jnp.cumsum(padded).astype(jnp.int32)])
    total = offp[E]
    gstart = jnp.arange(NTILES, dtype=jnp.int32) * TG
    te_raw = jnp.sum((offp[None, :E] <= gstart[:, None]).astype(jnp.int32),
                     axis=1) - 1
    te_last = jnp.sum((offp[:E] <= total - 1).astype(jnp.int32)) - 1
    act_fl = (gstart < total).astype(jnp.int32)
    te = jnp.where(act_fl > 0, te_raw, te_last)
    offp8 = offp[:E]

    grid_spec = pltpu.PrefetchScalarGridSpec(
        num_scalar_prefetch=3,
        grid=(NTILES, NI),
        in_specs=[
            pl.BlockSpec((N, H), lambda g, i, te, af, op: (0, 0)),
            pl.BlockSpec((N, E), lambda g, i, te, af, op: (0, 0)),
            pl.BlockSpec((N, 16), lambda g, i, te, af, op: (0, 0)),
            pl.BlockSpec((1, C, H), lambda g, i, te, af, op: (te[g], i, 0)),
            pl.BlockSpec((1, C, H), lambda g, i, te, af, op: (te[g], NI + i, 0)),
            pl.BlockSpec((1, 1, C), lambda g, i, te, af, op: (te[g] * 2 * NI + i, 0, 0)),
            pl.BlockSpec((1, 1, C), lambda g, i, te, af, op: (te[g] * 2 * NI + NI + i, 0, 0)),
            pl.BlockSpec((1, H, I), lambda g, i, te, af, op: (te[g], 0, 0)),
            pl.BlockSpec((1, 1, H), lambda g, i, te, af, op: (te[g], 0, 0)),
        ],
        out_specs=pl.BlockSpec((TG, H), lambda g, i, te, af, op: (g, 0)),
        scratch_shapes=[
            pltpu.VMEM((N, TG), jnp.bfloat16),
            pltpu.VMEM((TG, H), jnp.bfloat16),
            pltpu.VMEM((TG, 16), jnp.float32),
            pltpu.VMEM((TG, I), jnp.float32),
        ],
    )
    h2w = pl.pallas_call(
        _gemm_kernel,
        grid_spec=grid_spec,
        out_shape=jax.ShapeDtypeStruct((NP, H), jnp.bfloat16),
        compiler_params=pltpu.CompilerParams(
            dimension_semantics=("arbitrary", "arbitrary"),
            vmem_limit_bytes=64 * 1024 * 1024,
        ),
    )(te, act_fl, offp8, q1, rank, swt,
      w13_weight, w13_weight,
      w13_weight_scale.reshape(E * 2 * NI, 1, C),
      w13_weight_scale.reshape(E * 2 * NI, 1, C),
      w2_weight, w2_weight_scale.reshape(E, 1, H))

    grid_spec_c = pltpu.PrefetchScalarGridSpec(
        num_scalar_prefetch=1,
        grid=(N // TN, NP // PT),
        in_specs=[
            pl.BlockSpec((TN, E), lambda n, p, op: (n, 0)),
            pl.BlockSpec((PT, H), lambda n, p, op: (p, 0)),
        ],
        out_specs=pl.BlockSpec((TN, H), lambda n, p, op: (n, 0)),
    )
    out = pl.pallas_call(
        _combine_kernel,
        grid_spec=grid_spec_c,
        out_shape=jax.ShapeDtypeStruct((N, H), jnp.float32),
        compiler_params=pltpu.CompilerParams(
            dimension_semantics=("arbitrary", "arbitrary"),
            vmem_limit_bytes=64 * 1024 * 1024,
        ),
    )(offp8, rank, h2w)
    return out


def kernel(hidden_states, gate_weight, w13_weight, w13_weight_scale, w2_weight, w2_weight_scale):
    b, s, h = hidden_states.shape
    x = hidden_states.reshape(-1, h)
    out = _run(x, gate_weight, w13_weight, w13_weight_scale,
               w2_weight, w2_weight_scale)
    return out.reshape(b, s, h)


# speedup vs baseline: 2.3329x; 1.0012x over previous
import jax
import jax.numpy as jnp
from jax.experimental import pallas as pl
from jax.experimental.pallas import tpu as pltpu

B, S, H, I, E, K = 1, 2048, 2048, 4096, 8, 2
N = B * S

TG = 256
NTILES = N * K // TG + E
NP = NTILES * TG
C = 1024
NI = I // C
TN = 1024
PT = 2048
HUGE = 1 << 22


def _gate_kernel(x_ref, gw_ref, q1_ref, swt_ref, rank_ref, counts_ref):
    x = x_ref[...]
    logits = jax.lax.dot_general(
        x, gw_ref[...], (((1,), (1,)), ((), ())),
        preferred_element_type=jnp.float32)
    idx = jax.lax.broadcasted_iota(jnp.int32, (N, E), 1)
    m1 = jnp.max(logits, axis=1, keepdims=True)
    i1 = jnp.min(jnp.where(logits == m1, idx, E), axis=1, keepdims=True)
    masked = jnp.where(idx == i1, -jnp.inf, logits)
    m2 = jnp.max(masked, axis=1, keepdims=True)
    i2 = jnp.min(jnp.where(masked == m2, idx, E), axis=1, keepdims=True)
    b = jnp.exp(m2 - m1)
    denom = 1.0 + b
    wts = (jnp.where(idx == i1, 1.0 / denom, 0.0)
           + jnp.where(idx == i2, b / denom, 0.0))
    s1 = jnp.clip(jnp.max(jnp.abs(x), axis=1, keepdims=True), 1e-8, None) / 127.0
    q1_ref[...] = jnp.clip(jnp.round(x / s1), -127.0, 127.0).astype(jnp.bfloat16)
    zeros7 = jnp.zeros((N, 16 - E - 1), jnp.float32)
    swt_ref[...] = jnp.concatenate([wts, s1, zeros7], axis=1)

    m = (wts > 0.0).astype(jnp.float32)
    mb = m.astype(jnp.bfloat16)
    ck = 512
    for k in range(N // ck):
        ri = jax.lax.broadcasted_iota(jnp.int32, (ck, N), 0) + k * ck
        ci = jax.lax.broadcasted_iota(jnp.int32, (ck, N), 1)
        lk = (ci <= ri).astype(jnp.bfloat16)
        cum = jax.lax.dot_general(
            lk, mb, (((1,), (0,)), ((), ())),
            preferred_element_type=jnp.float32)
        mrows = m[k * ck:(k + 1) * ck]
        rank = (cum - mrows).astype(jnp.int32)
        rank_ref[k * ck:(k + 1) * ck, :] = jnp.where(mrows > 0.0, rank, HUGE)
        if k == N // ck - 1:
            counts_ref[...] = cum[ck - 1:ck, :].astype(jnp.int32)


def _gemm_kernel(te_ref, act_ref, offp_ref,
                 q1_ref, rank_ref, swt_ref, w13g_ref, w13u_ref,
                 s13g_ref, s13u_ref, w2_ref, s2w_ref, h2w_ref,
                 pt_s, q1t_s, st_s, act_s):
    g = pl.program_id(0)
    i = pl.program_id(1)
    te = te_ref[g]
    active = act_ref[g]

    @pl.when(active > 0)
    def _work():
        @pl.when(i == 0)
        def _gather():
            p0 = g * TG
            ci = jax.lax.broadcasted_iota(jnp.int32, (N, TG), 1) + p0
            acc = jnp.zeros((N, TG), jnp.float32)
            for e in range(E):
                pos_e = rank_ref[:, e:e + 1] + offp_ref[e]
                acc += (pos_e == ci).astype(jnp.float32)
            pt = acc.astype(jnp.bfloat16)
            pt_s[...] = pt
            q1t = jax.lax.dot_general(
                pt, q1_ref[...], (((0,), (0,)), ((), ())),
                preferred_element_type=jnp.float32)
            q1t_s[...] = q1t.astype(jnp.bfloat16)
            st_s[...] = jax.lax.dot_general(
                acc, swt_ref[...], (((0,), (0,)), ((), ())),
                precision=jax.lax.Precision.HIGHEST,
                preferred_element_type=jnp.float32)

        q1t = q1t_s[...]
        s1t = st_s[:, E:E + 1]
        hg = jax.lax.dot_general(
            q1t, w13g_ref[0].astype(jnp.bfloat16), (((1,), (1,)), ((), ())),
            preferred_element_type=jnp.float32) * s1t * s13g_ref[0]
        hu = jax.lax.dot_general(
            q1t, w13u_ref[0].astype(jnp.bfloat16), (((1,), (1,)), ((), ())),
            preferred_element_type=jnp.float32) * s1t * s13u_ref[0]
        act_s[:, pl.ds(i * C, C)] = hg * jax.lax.logistic(hg) * hu

        @pl.when(i == NI - 1)
        def _fc2():
            act = act_s[...]
            s2 = jnp.clip(jnp.max(jnp.abs(act), axis=1, keepdims=True),
                          1e-8, None) / 127.0
            q2 = jnp.clip(jnp.round(act / s2), -127.0, 127.0).astype(jnp.bfloat16)
            h2 = jax.lax.dot_general(
                q2, w2_ref[0].astype(jnp.bfloat16), (((1,), (1,)), ((), ())),
                preferred_element_type=jnp.float32) * s2 * s2w_ref[0]
            lidx = jax.lax.broadcasted_iota(jnp.int32, (TG, 16), 1)
            wcol = jnp.sum(jnp.where(lidx == te, st_s[...], 0.0),
                           axis=1, keepdims=True)
            h2w_ref[...] = (wcol * h2).astype(jnp.bfloat16)

    @pl.when(jnp.logical_and(active == 0, i == NI - 1))
    def _zero():
        h2w_ref[...] = jnp.zeros_like(h2w_ref)


def _combine_kernel(offp_ref, rank_ref, h2w_ref, out_ref):
    p = pl.program_id(1)
    p0 = p * PT
    ci = jax.lax.broadcasted_iota(jnp.int32, (TN, PT), 1) + p0
    acc = jnp.zeros((TN, PT), jnp.float32)
    for e in range(E):
        pos_e = rank_ref[:, e:e + 1] + offp_ref[e]
        acc += (pos_e == ci).astype(jnp.float32)
    gmat = acc.astype(jnp.bfloat16)
    contrib = jax.lax.dot_general(
        gmat, h2w_ref[...], (((1,), (0,)), ((), ())),
        preferred_element_type=jnp.float32)

    @pl.when(p == 0)
    def _init():
        out_ref[...] = contrib

    @pl.when(p != 0)
    def _acc():
        out_ref[...] += contrib


@jax.jit
def _run(x, gate_weight, w13_weight, w13_weight_scale, w2_weight, w2_weight_scale):
    q1, swt, rank, counts = pl.pallas_call(
        _gate_kernel,
        out_shape=[
            jax.ShapeDtypeStruct((N, H), jnp.bfloat16),
            jax.ShapeDtypeStruct((N, 16), jnp.float32),
            jax.ShapeDtypeStruct((N, E), jnp.int32),
            jax.ShapeDtypeStruct((1, E), jnp.int32),
        ],
        compiler_params=pltpu.CompilerParams(
            vmem_limit_bytes=64 * 1024 * 1024,
        ),
    )(x, gate_weight)

    cnt = counts.reshape(E)
    padded = ((cnt + TG - 1) // TG) * TG
    offp = jnp.concatenate([jnp.zeros((1,), jnp.int32),
                            jnp.cumsum(padded).astype(jnp.int32)])
    total = offp[E]
    gstart = jnp.arange(NTILES, dtype=jnp.int32) * TG
    te_raw = jnp.sum((offp[None, :E] <= gstart[:, None]).astype(jnp.int32),
                     axis=1) - 1
    te_last = jnp.sum((offp[:E] <= total - 1).astype(jnp.int32)) - 1
    act_fl = (gstart < total).astype(jnp.int32)
    te = jnp.where(act_fl > 0, te_raw, te_last)
    offp8 = offp[:E]

    grid_spec = pltpu.PrefetchScalarGridSpec(
        num_scalar_prefetch=3,
        grid=(NTILES, NI),
        in_specs=[
            pl.BlockSpec((N, H), lambda g, i, te, af, op: (0, 0)),
            pl.BlockSpec((N, E), lambda g, i, te, af, op: (0, 0)),
            pl.BlockSpec((N, 16), lambda g, i, te, af, op: (0, 0)),
            pl.BlockSpec((1, C, H), lambda g, i, te, af, op: (te[g], i, 0)),
            pl.BlockSpec((1, C, H), lambda g, i, te, af, op: (te[g], NI + i, 0)),
            pl.BlockSpec((1, 1, C), lambda g, i, te, af, op: (te[g] * 2 * NI + i, 0, 0)),
            pl.BlockSpec((1, 1, C), lambda g, i, te, af, op: (te[g] * 2 * NI + NI + i, 0, 0)),
            pl.BlockSpec((1, H, I), lambda g, i, te, af, op: (te[g], 0, 0)),
            pl.BlockSpec((1, 1, H), lambda g, i, te, af, op: (te[g], 0, 0)),
        ],
        out_specs=pl.BlockSpec((TG, H), lambda g, i, te, af, op: (g, 0)),
        scratch_shapes=[
            pltpu.VMEM((N, TG), jnp.bfloat16),
            pltpu.VMEM((TG, H), jnp.bfloat16),
            pltpu.VMEM((TG, 16), jnp.float32),
            pltpu.VMEM((TG, I), jnp.float32),
        ],
    )
    h2w = pl.pallas_call(
        _gemm_kernel,
        grid_spec=grid_spec,
        out_shape=jax.ShapeDtypeStruct((NP, H), jnp.bfloat16),
        compiler_params=pltpu.CompilerParams(
            dimension_semantics=("arbitrary", "arbitrary"),
            vmem_limit_bytes=64 * 1024 * 1024,
        ),
    )(te, act_fl, offp8, q1, rank, swt,
      w13_weight, w13_weight,
      w13_weight_scale.reshape(E * 2 * NI, 1, C),
      w13_weight_scale.reshape(E * 2 * NI, 1, C),
      w2_weight, w2_weight_scale.reshape(E, 1, H))

    grid_spec_c = pltpu.PrefetchScalarGridSpec(
        num_scalar_prefetch=1,
        grid=(N // TN, NP // PT),
        in_specs=[
            pl.BlockSpec((TN, E), lambda n, p, op: (n, 0)),
            pl.BlockSpec((PT, H), lambda n, p, op: (p, 0)),
        ],
        out_specs=pl.BlockSpec((TN, H), lambda n, p, op: (n, 0)),
    )
    out = pl.pallas_call(
        _combine_kernel,
        grid_spec=grid_spec_c,
        out_shape=jax.ShapeDtypeStruct((N, H), jnp.float32),
        compiler_params=pltpu.CompilerParams(
            dimension_semantics=("arbitrary", "arbitrary"),
            vmem_limit_bytes=64 * 1024 * 1024,
        ),
    )(offp8, rank, h2w)
    return out


def kernel(hidden_states, gate_weight, w13_weight, w13_weight_scale, w2_weight, w2_weight_scale):
    b, s, h = hidden_states.shape
    x = hidden_states.reshape(-1, h)
    out = _run(x, gate_weight, w13_weight, w13_weight_scale,
               w2_weight, w2_weight_scale)
    return out.reshape(b, s, h)
